# baseline (speedup 1.0000x reference)
import functools

import jax
import jax.numpy as jnp
from jax import lax
from jax.experimental import pallas as pl
from jax.experimental.pallas import tpu as pltpu
from jax.experimental.pallas import tpu_sc as plsc

_CH = 256
_NBUF = 6


def kernel(x, pos_table):
    seqlen = x.shape[1]
    _, dim = pos_table.shape

    info = plsc.get_sparse_core_info()
    nc = info.num_cores
    assert seqlen % nc == 0
    rows_per_sc = seqlen // nc
    ch = min(_CH, rows_per_sc)
    assert rows_per_sc % ch == 0
    nch = rows_per_sc // ch
    nbuf = min(_NBUF, nch)

    mesh = plsc.ScalarSubcoreMesh(axis_name="c", num_cores=nc)

    @functools.partial(
        pl.kernel,
        mesh=mesh,
        out_type=jax.ShapeDtypeStruct((seqlen, dim), pos_table.dtype),
        scratch_types=(
            [pltpu.VMEM_SHARED((ch, dim), pos_table.dtype)] * nbuf
            + [pltpu.SemaphoreType.DMA] * (2 * nbuf)
        ),
    )
    def copy_k(table_hbm, out_hbm, *scratch):
        bufs = scratch[:nbuf]
        isems = scratch[nbuf : 2 * nbuf]
        osems = scratch[2 * nbuf :]
        cid = lax.axis_index("c")
        base = cid * rows_per_sc

        in_h = [None] * nch
        out_h = [None] * nch
        for i in range(nbuf):
            in_h[i] = pltpu.async_copy(
                table_hbm.at[pl.ds(base + i * ch, ch)], bufs[i], isems[i]
            )
        for i in range(nch):
            b = i % nbuf
            in_h[i].wait()
            out_h[i] = pltpu.async_copy(
                bufs[b], out_hbm.at[pl.ds(base + i * ch, ch)], osems[b]
            )
            if i + nbuf < nch:
                out_h[i].wait()
                in_h[i + nbuf] = pltpu.async_copy(
                    table_hbm.at[pl.ds(base + (i + nbuf) * ch, ch)], bufs[b], isems[b]
                )
        for i in range(max(0, nch - nbuf), nch):
            out_h[i].wait()

    return copy_k(pos_table)


# speedup vs baseline: 1.0127x; 1.0127x over previous
import functools

import jax
import jax.numpy as jnp
from jax import lax
from jax.experimental import pallas as pl
from jax.experimental.pallas import tpu as pltpu
from jax.experimental.pallas import tpu_sc as plsc

_CH = 32
_NBUF = 2


def kernel(x, pos_table):
    seqlen = x.shape[1]
    _, dim = pos_table.shape

    info = plsc.get_sparse_core_info()
    nc, ns = info.num_cores, info.num_subcores
    nw = nc * ns
    assert seqlen % nw == 0
    rows_per_w = seqlen // nw
    ch = min(_CH, rows_per_w)
    assert rows_per_w % ch == 0
    nch = rows_per_w // ch
    nbuf = min(_NBUF, nch)
    assert nch % nbuf == 0
    ngroups = nch // nbuf

    mesh = plsc.VectorSubcoreMesh(core_axis_name="c", subcore_axis_name="s")

    @functools.partial(
        pl.kernel,
        mesh=mesh,
        out_type=jax.ShapeDtypeStruct((seqlen, dim), pos_table.dtype),
        scratch_types=(
            [pltpu.VMEM((ch, dim), pos_table.dtype)] * nbuf
            + [pltpu.SemaphoreType.DMA] * (2 * nbuf)
        ),
    )
    def copy_k(table_hbm, out_hbm, *scratch):
        bufs = scratch[:nbuf]
        isems = scratch[nbuf : 2 * nbuf]
        osems = scratch[2 * nbuf :]
        wid = lax.axis_index("s") * nc + lax.axis_index("c")
        base = wid * rows_per_w

        def in_start(b, chunk):
            pltpu.make_async_copy(
                table_hbm.at[pl.ds(base + chunk * ch, ch)], bufs[b], isems[b]
            ).start()

        def in_wait(b):
            pltpu.make_async_copy(
                table_hbm.at[pl.ds(base, ch)], bufs[b], isems[b]
            ).wait()

        def out_start(b, chunk):
            pltpu.make_async_copy(
                bufs[b], out_hbm.at[pl.ds(base + chunk * ch, ch)], osems[b]
            ).start()

        def out_wait(b):
            pltpu.make_async_copy(
                bufs[b], out_hbm.at[pl.ds(base, ch)], osems[b]
            ).wait()

        for b in range(nbuf):
            in_start(b, b)

        def group(g, carry):
            c0 = g * nbuf
            for b in range(nbuf):
                in_wait(b)
                out_start(b, c0 + b)
            for b in range(nbuf):
                out_wait(b)

                @pl.when(g + 1 < ngroups)
                def _():
                    in_start(b, c0 + nbuf + b)

            return carry

        lax.fori_loop(0, ngroups, group, 0)

    return copy_k(pos_table)


# speedup vs baseline: 1.0181x; 1.0054x over previous
import functools

import jax
import jax.numpy as jnp
from jax import lax
from jax.experimental import pallas as pl
from jax.experimental.pallas import tpu as pltpu
from jax.experimental.pallas import tpu_sc as plsc

_K = 1024
_BLK = 512
_CH = 16
_NBUF = 2


def _sc_copy(pos_table, k_rows):
    _, dim = pos_table.shape
    info = plsc.get_sparse_core_info()
    nc, ns = info.num_cores, info.num_subcores
    nw = nc * ns
    rows_per_w = k_rows // nw
    ch = min(_CH, rows_per_w)
    nch = rows_per_w // ch
    nbuf = min(_NBUF, nch)

    mesh = plsc.VectorSubcoreMesh(core_axis_name="c", subcore_axis_name="s")

    @functools.partial(
        pl.kernel,
        mesh=mesh,
        out_type=jax.ShapeDtypeStruct((k_rows, dim), pos_table.dtype),
        scratch_types=(
            [pltpu.VMEM((ch, dim), pos_table.dtype)] * nbuf
            + [pltpu.SemaphoreType.DMA] * (2 * nbuf)
        ),
    )
    def copy_k(table_hbm, out_hbm, *scratch):
        bufs = scratch[:nbuf]
        isems = scratch[nbuf : 2 * nbuf]
        osems = scratch[2 * nbuf :]
        wid = lax.axis_index("s") * nc + lax.axis_index("c")
        base = wid * rows_per_w

        in_h = [None] * nch
        out_h = [None] * nch
        for i in range(nbuf):
            in_h[i] = pltpu.async_copy(
                table_hbm.at[pl.ds(base + i * ch, ch)], bufs[i], isems[i]
            )
        for i in range(nch):
            b = i % nbuf
            in_h[i].wait()
            out_h[i] = pltpu.async_copy(
                bufs[b], out_hbm.at[pl.ds(base + i * ch, ch)], osems[b]
            )
            if i + nbuf < nch:
                out_h[i].wait()
                in_h[i + nbuf] = pltpu.async_copy(
                    table_hbm.at[pl.ds(base + (i + nbuf) * ch, ch)], bufs[b], isems[b]
                )
        for i in range(max(0, nch - nbuf), nch):
            out_h[i].wait()

    return copy_k(pos_table)


def kernel(x, pos_table):
    seqlen = x.shape[1]
    nrows, dim = pos_table.shape
    k = _K

    sc_part = _sc_copy(pos_table, k)

    blk_off = k // _BLK

    def body(t_ref, o_ref):
        o_ref[...] = t_ref[...]

    tc_full = pl.pallas_call(
        body,
        grid=((seqlen - k) // _BLK,),
        in_specs=[pl.BlockSpec((_BLK, dim), lambda i: (i + blk_off, 0))],
        out_specs=pl.BlockSpec((_BLK, dim), lambda i: (i + blk_off, 0)),
        out_shape=jax.ShapeDtypeStruct((seqlen, dim), pos_table.dtype),
    )(pos_table)

    return lax.dynamic_update_slice(tc_full, sc_part, (0, 0))


# speedup vs baseline: 1.0935x; 1.0740x over previous
import functools

import jax
import jax.numpy as jnp
from jax import lax
from jax.experimental import pallas as pl
from jax.experimental.pallas import tpu as pltpu
from jax.experimental.pallas import tpu_sc as plsc

_CHUNK_ROWS = 32
_NBUF = 3


def kernel(x, pos_table):
    seqlen = x.shape[1]
    _, dim = pos_table.shape

    info = plsc.get_sparse_core_info()
    nc, ns = info.num_cores, info.num_subcores
    nw = nc * ns
    assert seqlen % nw == 0
    rows_per_w = seqlen // nw
    ch = min(_CHUNK_ROWS, rows_per_w)
    assert rows_per_w % ch == 0
    nch = rows_per_w // ch
    nbuf = min(_NBUF, nch)

    mesh = plsc.VectorSubcoreMesh(core_axis_name="c", subcore_axis_name="s")

    @functools.partial(
        pl.kernel,
        mesh=mesh,
        out_type=jax.ShapeDtypeStruct((seqlen, dim), pos_table.dtype),
        scratch_types=(
            [pltpu.VMEM((ch, dim), pos_table.dtype)] * nbuf
            + [pltpu.SemaphoreType.DMA] * (2 * nbuf)
        ),
    )
    def copy_k(table_hbm, out_hbm, *scratch):
        bufs = scratch[:nbuf]
        isems = scratch[nbuf : 2 * nbuf]
        osems = scratch[2 * nbuf :]
        wid = lax.axis_index("c") * ns + lax.axis_index("s")
        base = wid * rows_per_w

        in_h = [None] * nch
        out_h = [None] * nch
        for i in range(nbuf):
            in_h[i] = pltpu.async_copy(
                table_hbm.at[pl.ds(base + i * ch, ch)], bufs[i % nbuf], isems[i % nbuf]
            )
        for i in range(nch):
            b = i % nbuf
            in_h[i].wait()
            out_h[i] = pltpu.async_copy(
                bufs[b], out_hbm.at[pl.ds(base + i * ch, ch)], osems[b]
            )
            if i + nbuf < nch:
                out_h[i].wait()
                in_h[i + nbuf] = pltpu.async_copy(
                    table_hbm.at[pl.ds(base + (i + nbuf) * ch, ch)], bufs[b], isems[b]
                )
        for i in range(max(0, nch - nbuf), nch):
            out_h[i].wait()

    return copy_k(pos_table)
